# baseline (speedup 1.0000x reference)
import functools

import jax
import jax.numpy as jnp
from jax import lax
from jax.experimental import pallas as pl
from jax.experimental.pallas import tpu as pltpu
from jax.experimental.pallas import tpu_sc as plsc

_LANES = 16
_NBUF = 3


def _pos_encoding(seq_len, d_model):
    pos = jnp.arange(seq_len, dtype=jnp.float32)[:, None]
    i = jnp.arange(0, d_model, 2, dtype=jnp.float32)[None, :]
    angle = pos / jnp.power(10000.0, i / d_model)
    pe = jnp.zeros((seq_len, d_model), dtype=jnp.float32)
    pe = pe.at[:, 0::2].set(jnp.sin(angle))
    pe = pe.at[:, 1::2].set(jnp.cos(angle))
    return pe


@functools.partial(jax.jit, static_argnums=(3, 4, 5))
def _sc_embed(xf, pe, table, batch, seq, d):
    info = plsc.get_sparse_core_info()
    nc, ns = info.num_cores, info.num_subcores
    nw = nc * ns
    s_per_w = seq // nw
    k = 16
    nchunk = s_per_w // k
    nsteps = nchunk * batch
    ncol = d // _LANES

    mesh = plsc.VectorSubcoreMesh(core_axis_name="c", subcore_axis_name="s")

    @functools.partial(
        pl.kernel,
        out_type=jax.ShapeDtypeStruct((batch * seq, d), jnp.float32),
        mesh=mesh,
        scratch_types=[
            pltpu.VMEM((_NBUF, k), jnp.int32),
            pltpu.VMEM((k, d), jnp.float32),
            pltpu.VMEM((k, d), jnp.float32),
            pltpu.VMEM((k, d), jnp.float32),
            pltpu.VMEM((k, d), jnp.float32),
            pltpu.SemaphoreType.DMA,
            pltpu.SemaphoreType.DMA,
            pltpu.SemaphoreType.DMA,
            pltpu.SemaphoreType.DMA,
            pltpu.SemaphoreType.DMA,
            pltpu.SemaphoreType.DMA,
        ],
    )
    def run(x_hbm, pe_hbm, table_hbm, out_hbm,
            idx_v, pe_v, r0, r1, r2, g0, g1, g2, o0, o1, o2):
        rows = [r0, r1, r2]
        gsem = [g0, g1, g2]
        osem = [o0, o1, o2]
        wid = lax.axis_index("s") * nc + lax.axis_index("c")
        w0 = wid * s_per_w

        def off_of(t):
            c, b = divmod(t, batch)
            return b * seq + w0 + c * k

        def fire_gather(t):
            p = t % _NBUF
            pltpu.sync_copy(x_hbm.at[pl.ds(off_of(t), k)], idx_v.at[p])
            pltpu.async_copy(table_hbm.at[idx_v.at[p]], rows[p], gsem[p])

        def wait_gather(t):
            p = t % _NBUF
            pltpu.make_async_copy(
                table_hbm.at[idx_v.at[p]], rows[p], gsem[p]).wait()

        def wait_store(t):
            p = t % _NBUF
            pltpu.make_async_copy(
                rows[p], out_hbm.at[pl.ds(off_of(t), k)], osem[p]).wait()

        def add_pe(p):
            buf = rows[p]

            def row_body(r, carry):
                def col_body(j, carry2):
                    col = j * _LANES
                    buf[r, pl.ds(col, _LANES)] = (
                        buf[r, pl.ds(col, _LANES)]
                        + pe_v[r, pl.ds(col, _LANES)]
                    )
                    return carry2

                lax.fori_loop(0, ncol, col_body, 0, unroll=8)
                return carry

            lax.fori_loop(0, k, row_body, 0)

        pltpu.sync_copy(pe_hbm.at[pl.ds(w0, k)], pe_v)
        fire_gather(0)
        fire_gather(1)

        for t in range(nsteps):
            p = t % _NBUF
            wait_gather(t)
            if t % batch == 0 and t > 0:
                c = t // batch
                pltpu.sync_copy(pe_hbm.at[pl.ds(w0 + c * k, k)], pe_v)
            add_pe(p)
            pltpu.async_copy(
                rows[p], out_hbm.at[pl.ds(off_of(t), k)], osem[p])
            if t + 2 < nsteps:
                if t >= 1:
                    wait_store(t - 1)
                fire_gather(t + 2)

        for t in (nsteps - 3, nsteps - 2, nsteps - 1):
            wait_store(t)

    return run(xf, pe, table)


def kernel(x, table):
    b, s = x.shape
    v, d = table.shape
    xf = x.reshape(-1).astype(jnp.int32)
    pe = _pos_encoding(s, d)
    out = _sc_embed(xf, pe, table, b, s, d)
    return out.reshape(b, s, d)


# speedup vs baseline: 1.5964x; 1.5964x over previous
import functools

import jax
import jax.numpy as jnp
from jax import lax
from jax.experimental import pallas as pl
from jax.experimental.pallas import tpu as pltpu
from jax.experimental.pallas import tpu_sc as plsc

_LANES = 16
_NBUF = 3


def _pos_encoding(seq_len, d_model):
    pos = jnp.arange(seq_len, dtype=jnp.float32)[:, None]
    i = jnp.arange(0, d_model, 2, dtype=jnp.float32)[None, :]
    angle = pos / jnp.power(10000.0, i / d_model)
    pe = jnp.zeros((seq_len, d_model), dtype=jnp.float32)
    pe = pe.at[:, 0::2].set(jnp.sin(angle))
    pe = pe.at[:, 1::2].set(jnp.cos(angle))
    return pe


@functools.partial(jax.jit, static_argnums=(3, 4, 5))
def _sc_embed(xf, pe, table, batch, seq, d):
    info = plsc.get_sparse_core_info()
    nc, ns = info.num_cores, info.num_subcores
    nw = nc * ns
    s_per_w = seq // nw
    k = 16
    nchunk = s_per_w // k
    nsteps = nchunk * batch
    ncol = d // _LANES

    mesh = plsc.VectorSubcoreMesh(core_axis_name="c", subcore_axis_name="s")

    @functools.partial(
        pl.kernel,
        out_type=jax.ShapeDtypeStruct((batch * seq, d), jnp.float32),
        mesh=mesh,
        scratch_types=[
            pltpu.VMEM((_NBUF, k), jnp.int32),
            pltpu.VMEM((k, d), jnp.float32),
            pltpu.VMEM((k, d), jnp.float32),
            pltpu.VMEM((k, d), jnp.float32),
            pltpu.VMEM((k, d), jnp.float32),
            pltpu.SemaphoreType.DMA,
            pltpu.SemaphoreType.DMA,
            pltpu.SemaphoreType.DMA,
            pltpu.SemaphoreType.DMA,
            pltpu.SemaphoreType.DMA,
            pltpu.SemaphoreType.DMA,
        ],
    )
    def run(x_hbm, pe_hbm, table_hbm, out_hbm,
            idx_v, pe_v, r0, r1, r2, g0, g1, g2, o0, o1, o2):
        rows = [r0, r1, r2]
        gsem = [g0, g1, g2]
        osem = [o0, o1, o2]
        wid = lax.axis_index("s") * nc + lax.axis_index("c")
        w0 = wid * s_per_w

        def off_of(t):
            c, b = divmod(t, batch)
            return b * seq + w0 + c * k

        def fire_gather(t):
            p = t % _NBUF
            pltpu.sync_copy(x_hbm.at[pl.ds(off_of(t), k)], idx_v.at[p])
            pltpu.async_copy(table_hbm.at[idx_v.at[p]], rows[p], gsem[p])

        def wait_gather(t):
            p = t % _NBUF
            pltpu.make_async_copy(
                table_hbm.at[idx_v.at[p]], rows[p], gsem[p]).wait()

        def wait_store(t):
            p = t % _NBUF
            pltpu.make_async_copy(
                rows[p], out_hbm.at[pl.ds(off_of(t), k)], osem[p]).wait()

        def add_pe(p):
            buf = rows[p]

            def body(i, carry):
                r = i // ncol
                col = (i % ncol) * _LANES
                buf[r, pl.ds(col, _LANES)] = (
                    buf[r, pl.ds(col, _LANES)] + pe_v[r, pl.ds(col, _LANES)]
                )
                return carry

            lax.fori_loop(0, k * ncol, body, 0, unroll=4)

        pltpu.sync_copy(pe_hbm.at[pl.ds(w0, k)], pe_v)
        fire_gather(0)
        fire_gather(1)

        for t in range(nsteps):
            p = t % _NBUF
            wait_gather(t)
            if t % batch == 0 and t > 0:
                c = t // batch
                pltpu.sync_copy(pe_hbm.at[pl.ds(w0 + c * k, k)], pe_v)
            add_pe(p)
            pltpu.async_copy(
                rows[p], out_hbm.at[pl.ds(off_of(t), k)], osem[p])
            if t + 2 < nsteps:
                if t >= 1:
                    wait_store(t - 1)
                fire_gather(t + 2)

        for t in (nsteps - 3, nsteps - 2, nsteps - 1):
            wait_store(t)

    return run(xf, pe, table)


def kernel(x, table):
    b, s = x.shape
    v, d = table.shape
    xf = x.reshape(-1).astype(jnp.int32)
    pe = _pos_encoding(s, d)
    out = _sc_embed(xf, pe, table, b, s, d)
    return out.reshape(b, s, d)


# speedup vs baseline: 2.9477x; 1.8465x over previous
import functools

import numpy as np

import jax
import jax.numpy as jnp
from jax import lax
from jax.experimental import pallas as pl
from jax.experimental.pallas import tpu as pltpu
from jax.experimental.pallas import tpu_sc as plsc

_LANES = 16
_NBUF = 3

_PE_CACHE = {}


def _pos_encoding(seq_len, d_model):
    key = (seq_len, d_model)
    if key not in _PE_CACHE:
        pos = np.arange(seq_len, dtype=np.float32)[:, None]
        i = np.arange(0, d_model, 2, dtype=np.float32)[None, :]
        angle = pos / np.power(np.float32(10000.0), i / np.float32(d_model))
        pe = np.empty((seq_len, d_model), dtype=np.float32)
        pe[:, 0::2] = np.sin(angle)
        pe[:, 1::2] = np.cos(angle)
        _PE_CACHE[key] = jnp.asarray(pe)
    return _PE_CACHE[key]


@functools.partial(jax.jit, static_argnums=(3, 4, 5))
def _sc_embed(x, pe, table, batch, seq, d):
    info = plsc.get_sparse_core_info()
    nc, ns = info.num_cores, info.num_subcores
    nw = nc * ns
    s_per_w = seq // nw
    k = 16
    nchunk = s_per_w // k
    nsteps = nchunk * batch
    ncol = d // _LANES

    mesh = plsc.VectorSubcoreMesh(core_axis_name="c", subcore_axis_name="s")

    @functools.partial(
        pl.kernel,
        out_type=jax.ShapeDtypeStruct((batch, seq, d), jnp.float32),
        mesh=mesh,
        scratch_types=[
            pltpu.VMEM((_NBUF, k), jnp.int32),
            pltpu.VMEM((k, d), jnp.float32),
            pltpu.VMEM((k, d), jnp.float32),
            pltpu.VMEM((k, d), jnp.float32),
            pltpu.VMEM((k, d), jnp.float32),
            pltpu.SemaphoreType.DMA,
            pltpu.SemaphoreType.DMA,
            pltpu.SemaphoreType.DMA,
            pltpu.SemaphoreType.DMA,
            pltpu.SemaphoreType.DMA,
            pltpu.SemaphoreType.DMA,
        ],
    )
    def run(x_hbm, pe_hbm, table_hbm, out_hbm,
            idx_v, pe_v, r0, r1, r2, g0, g1, g2, o0, o1, o2):
        rows = [r0, r1, r2]
        gsem = [g0, g1, g2]
        osem = [o0, o1, o2]
        wid = lax.axis_index("s") * nc + lax.axis_index("c")
        w0 = wid * s_per_w

        def cb_of(t):
            c, b = divmod(t, batch)
            return c, b

        def fire_gather(t):
            p = t % _NBUF
            c, b = cb_of(t)
            pltpu.sync_copy(x_hbm.at[b, pl.ds(w0 + c * k, k)], idx_v.at[p])
            pltpu.async_copy(table_hbm.at[idx_v.at[p]], rows[p], gsem[p])

        def wait_gather(t):
            p = t % _NBUF
            pltpu.make_async_copy(
                table_hbm.at[idx_v.at[p]], rows[p], gsem[p]).wait()

        def fire_store(t):
            p = t % _NBUF
            c, b = cb_of(t)
            pltpu.async_copy(
                rows[p], out_hbm.at[b, pl.ds(w0 + c * k, k)], osem[p])

        def wait_store(t):
            p = t % _NBUF
            c, b = cb_of(t)
            pltpu.make_async_copy(
                rows[p], out_hbm.at[b, pl.ds(w0 + c * k, k)], osem[p]).wait()

        def add_pe(p):
            buf = rows[p]

            def body(i, carry):
                r = i // ncol
                col = (i % ncol) * _LANES
                buf[r, pl.ds(col, _LANES)] = (
                    buf[r, pl.ds(col, _LANES)] + pe_v[r, pl.ds(col, _LANES)]
                )
                return carry

            lax.fori_loop(0, k * ncol, body, 0, unroll=4)

        pltpu.sync_copy(pe_hbm.at[pl.ds(w0, k)], pe_v)
        fire_gather(0)
        fire_gather(1)

        for t in range(nsteps):
            p = t % _NBUF
            wait_gather(t)
            if t % batch == 0 and t > 0:
                c = t // batch
                pltpu.sync_copy(pe_hbm.at[pl.ds(w0 + c * k, k)], pe_v)
            add_pe(p)
            fire_store(t)
            if t + 2 < nsteps:
                if t >= 1:
                    wait_store(t - 1)
                fire_gather(t + 2)

        for t in (nsteps - 3, nsteps - 2, nsteps - 1):
            wait_store(t)

    return run(x, pe, table)


def kernel(x, table):
    b, s = x.shape
    v, d = table.shape
    pe = _pos_encoding(s, d)
    return _sc_embed(x.astype(jnp.int32), pe, table, b, s, d)


# speedup vs baseline: 2.9530x; 1.0018x over previous
import functools

import numpy as np

import jax
import jax.numpy as jnp
from jax import lax
from jax.experimental import pallas as pl
from jax.experimental.pallas import tpu as pltpu
from jax.experimental.pallas import tpu_sc as plsc

_LANES = 16
_NBUF = 3

_PE_CACHE = {}


def _pos_encoding(seq_len, d_model):
    key = (seq_len, d_model)
    if key not in _PE_CACHE:
        pos = np.arange(seq_len, dtype=np.float32)[:, None]
        i = np.arange(0, d_model, 2, dtype=np.float32)[None, :]
        angle = pos / np.power(np.float32(10000.0), i / np.float32(d_model))
        pe = np.empty((seq_len, d_model), dtype=np.float32)
        pe[:, 0::2] = np.sin(angle)
        pe[:, 1::2] = np.cos(angle)
        _PE_CACHE[key] = jnp.asarray(pe)
    return _PE_CACHE[key]


@functools.partial(jax.jit, static_argnums=(3, 4, 5))
def _sc_embed(x, pe, table, batch, seq, d):
    info = plsc.get_sparse_core_info()
    nc, ns = info.num_cores, info.num_subcores
    nw = nc * ns
    s_per_w = seq // nw
    k = 16
    nchunk = s_per_w // k
    nsteps = nchunk * batch
    ncol = d // _LANES

    mesh = plsc.VectorSubcoreMesh(core_axis_name="c", subcore_axis_name="s")

    @functools.partial(
        pl.kernel,
        out_type=jax.ShapeDtypeStruct((batch, seq, d), jnp.float32),
        mesh=mesh,
        scratch_types=[
            pltpu.VMEM((_NBUF, k), jnp.int32),
            pltpu.VMEM((k, d), jnp.float32),
            pltpu.VMEM((k, d), jnp.float32),
            pltpu.VMEM((k, d), jnp.float32),
            pltpu.VMEM((k, d), jnp.float32),
            pltpu.SemaphoreType.DMA,
            pltpu.SemaphoreType.DMA,
            pltpu.SemaphoreType.DMA,
            pltpu.SemaphoreType.DMA,
            pltpu.SemaphoreType.DMA,
            pltpu.SemaphoreType.DMA,
        ],
    )
    def run(x_hbm, pe_hbm, table_hbm, out_hbm,
            idx_v, pe_v, r0, r1, r2, g0, g1, g2, o0, o1, o2):
        rows = [r0, r1, r2]
        gsem = [g0, g1, g2]
        osem = [o0, o1, o2]
        wid = lax.axis_index("s") * nc + lax.axis_index("c")
        w0 = wid * s_per_w

        def cb_of(t):
            c, b = divmod(t, batch)
            return c, b

        def fire_gather(t):
            p = t % _NBUF
            c, b = cb_of(t)
            pltpu.sync_copy(x_hbm.at[b, pl.ds(w0 + c * k, k)], idx_v.at[p])
            pltpu.async_copy(table_hbm.at[idx_v.at[p]], rows[p], gsem[p])

        def wait_gather(t):
            p = t % _NBUF
            pltpu.make_async_copy(
                table_hbm.at[idx_v.at[p]], rows[p], gsem[p]).wait()

        def fire_store(t):
            p = t % _NBUF
            c, b = cb_of(t)
            pltpu.async_copy(
                rows[p], out_hbm.at[b, pl.ds(w0 + c * k, k)], osem[p])

        def wait_store(t):
            p = t % _NBUF
            c, b = cb_of(t)
            pltpu.make_async_copy(
                rows[p], out_hbm.at[b, pl.ds(w0 + c * k, k)], osem[p]).wait()

        def add_pe(p):
            buf = rows[p]

            def body(i, carry):
                r = i // ncol
                col = (i % ncol) * _LANES
                plsc.addupdate(
                    buf.at[r, pl.ds(col, _LANES)],
                    pe_v[r, pl.ds(col, _LANES)],
                )
                return carry

            lax.fori_loop(0, k * ncol, body, 0, unroll=4)

        pltpu.sync_copy(pe_hbm.at[pl.ds(w0, k)], pe_v)
        fire_gather(0)
        fire_gather(1)

        for t in range(nsteps):
            p = t % _NBUF
            wait_gather(t)
            if t % batch == 0 and t > 0:
                c = t // batch
                pltpu.sync_copy(pe_hbm.at[pl.ds(w0 + c * k, k)], pe_v)
            add_pe(p)
            fire_store(t)
            if t + 2 < nsteps:
                if t >= 1:
                    wait_store(t - 1)
                fire_gather(t + 2)

        for t in (nsteps - 3, nsteps - 2, nsteps - 1):
            wait_store(t)

    return run(x, pe, table)


def kernel(x, table):
    b, s = x.shape
    v, d = table.shape
    pe = _pos_encoding(s, d)
    return _sc_embed(x.astype(jnp.int32), pe, table, b, s, d)
